# baseline (speedup 1.0000x reference)
import jax
import jax.numpy as jnp
from jax import lax
from jax.experimental import pallas as pl
from jax.experimental.pallas import tpu as pltpu
from jax.experimental.pallas import tpu_sc as plsc

_GRID = 32
_NPATCH = _GRID * _GRID
_L = 16
_NPTS = 320
_CHUNKS = ((0, 8), (8, 12), (12, 20))


def _sc_body(xs_hbm, ys_hbm, ew_hbm, nw_hbm, mw_hbm, out_hbm,
             xs_v, ys_v, ew_v, nw_v, mw_v,
             m_e, m_n, m_m, out_v):
    first = (lax.axis_index("c") == 0) & (lax.axis_index("s") == 0)

    @pl.when(first)
    def _():
        pltpu.sync_copy(xs_hbm, xs_v)
        pltpu.sync_copy(ys_hbm, ys_v)
        pltpu.sync_copy(ew_hbm, ew_v)
        pltpu.sync_copy(nw_hbm, nw_v)
        pltpu.sync_copy(mw_hbm, mw_v)

        zeros_f = jnp.zeros((_L,), jnp.float32)
        for j in range(_NPATCH // _L):
            sl = pl.ds(j * _L, _L)
            m_e[sl] = zeros_f
            m_n[sl] = zeros_f
            m_m[sl] = zeros_f

        one_f = jnp.ones((_L,), jnp.float32)
        cap = jnp.full((_L,), _GRID - 1, jnp.int32)

        for mask, (c0, c1) in zip((m_e, m_n, m_m), _CHUNKS):
            for i in range(c0, c1):
                sl = pl.ds(i * _L, _L)
                xs = xs_v[sl]
                ys = ys_v[sl]
                r = jnp.minimum(ys >> 4, cap)
                c = jnp.minimum(xs >> 4, cap)
                bins = (r << 5) | c
                plsc.store_scatter(mask, [bins], one_f)

        for j in range(_NPATCH // _L):
            sl = pl.ds(j * _L, _L)
            acc = (ew_v[sl] - one_f) * m_e[sl] + one_f
            acc = acc + (nw_v[sl] - one_f) * m_n[sl]
            out_v[sl] = acc + (mw_v[sl] - one_f) * m_m[sl]

        pltpu.sync_copy(out_v, out_hbm)


def kernel(eye_landmarks, nose_landmarks, mouth_landmarks,
           eye_weight, nose_weight, mouth_weight):
    eye = eye_landmarks.astype(jnp.int32)
    nose = nose_landmarks.astype(jnp.int32)
    mouth = mouth_landmarks.astype(jnp.int32)
    xs = jnp.concatenate([eye[:, 0], nose[:, 0], mouth[:, 0]])
    ys = jnp.concatenate([eye[:, 1], nose[:, 1], mouth[:, 1]])

    mesh = plsc.VectorSubcoreMesh(core_axis_name="c", subcore_axis_name="s",
                                  num_cores=1)
    run = pl.kernel(
        _sc_body,
        out_type=jax.ShapeDtypeStruct((_NPATCH,), jnp.float32),
        mesh=mesh,
        compiler_params=pltpu.CompilerParams(needs_layout_passes=False),
        scratch_types=[
            pltpu.VMEM((_NPTS,), jnp.int32),
            pltpu.VMEM((_NPTS,), jnp.int32),
            pltpu.VMEM((_NPATCH,), jnp.float32),
            pltpu.VMEM((_NPATCH,), jnp.float32),
            pltpu.VMEM((_NPATCH,), jnp.float32),
            pltpu.VMEM((_NPATCH,), jnp.float32),
            pltpu.VMEM((_NPATCH,), jnp.float32),
            pltpu.VMEM((_NPATCH,), jnp.float32),
            pltpu.VMEM((_NPATCH,), jnp.float32),
        ],
    )
    return run(xs, ys, eye_weight, nose_weight, mouth_weight)
